# Initial kernel scaffold; baseline (speedup 1.0000x reference)
import jax
import jax.numpy as jnp
from jax.experimental import pallas as pl


def kernel(original, generated):
    raise NotImplementedError("write your pallas kernel here")


# speedup vs baseline: 48.6225x; 48.6225x over previous
import functools

import jax
import jax.numpy as jnp
from jax import lax
from jax.experimental import pallas as pl
from jax.experimental.pallas import tpu as pltpu
from jax.experimental.pallas import tpu_sc as plsc

LEVELS = 256
H = 512
W = 512
N_IMG = 8
NBINS = LEVELS * LEVELS
SHIFTS = ((1, 0), (1, 1), (0, 1), (-1, 1))

NWORKERS = 32
PIX_PER_TENSOR = N_IMG * H * W
TOTAL_V = 2 * 4 * PIX_PER_TENSOR
PER_WORKER = TOTAL_V // NWORKERS
CHUNK = 16384
NCHUNK = PER_WORKER // CHUNK


def _roll2(x, dx, dy):
    if dx == 1:
        x = jnp.concatenate([x[-1:, :], x[:-1, :]], axis=0)
    elif dx == -1:
        x = jnp.concatenate([x[1:, :], x[:1, :]], axis=0)
    if dy == 1:
        x = jnp.concatenate([x[:, -1:], x[:, :-1]], axis=1)
    return x


def _joint_body(o_ref, g_ref, v_ref):
    for t, ref in enumerate((o_ref, g_ref)):
        mean = (ref[0, 0] + ref[0, 1] + ref[0, 2]) * jnp.float32(1.0 / 3.0)
        gray = jnp.round(mean * float(LEVELS - 1)).astype(jnp.int32)
        for k, (dx, dy) in enumerate(SHIFTS):
            q = _roll2(gray, dx, dy)
            v_ref[t, k, 0] = gray * LEVELS + q


def _joint(original, generated):
    return pl.pallas_call(
        _joint_body,
        grid=(N_IMG,),
        in_specs=[
            pl.BlockSpec((1, 3, H, W), lambda n: (n, 0, 0, 0)),
            pl.BlockSpec((1, 3, H, W), lambda n: (n, 0, 0, 0)),
        ],
        out_specs=pl.BlockSpec((2, 4, 1, H, W), lambda n: (0, 0, n, 0, 0)),
        out_shape=jax.ShapeDtypeStruct((2, 4, N_IMG, H, W), jnp.int32),
    )(original, generated)


def _hist_body(v_hbm, out_hbm, hist, buf0, buf1, sem0, sem1):
    wid = lax.axis_index("s") * 2 + lax.axis_index("c")
    bufs = (buf0, buf1)
    sems = (sem0, sem1)

    zeros16 = jnp.zeros((16,), jnp.int32)
    ones16 = jnp.full((16,), 1, jnp.int32)

    def zbody(i, carry):
        hist[pl.ds(i * 16, 16)] = zeros16
        return carry

    lax.fori_loop(0, NBINS // 16, zbody, 0, unroll=8)

    pltpu.async_copy(v_hbm.at[wid, pl.ds(0, CHUNK)], bufs[0], sems[0])

    def pair_body(p, carry):
        for s in range(2):
            c = p * 2 + s
            nxt = c + 1

            @pl.when(nxt < NCHUNK)
            def _():
                pltpu.async_copy(
                    v_hbm.at[wid, pl.ds(nxt * CHUNK, CHUNK)],
                    bufs[1 - s],
                    sems[1 - s],
                )

            pltpu.make_async_copy(
                v_hbm.at[wid, pl.ds(c * CHUNK, CHUNK)], bufs[s], sems[s]
            ).wait()

            buf_s = bufs[s]

            def scat(j, carry2):
                idx = buf_s[pl.ds(j * 16, 16)]
                plsc.addupdate_scatter(hist, [idx], ones16)
                return carry2

            lax.fori_loop(0, CHUNK // 16, scat, 0, unroll=8)
        return carry

    lax.fori_loop(0, NCHUNK // 2, pair_body, 0)

    pltpu.sync_copy(hist, out_hbm.at[wid])


def _hist(v32):
    mesh = plsc.VectorSubcoreMesh(core_axis_name="c", subcore_axis_name="s")
    f = functools.partial(
        pl.kernel,
        out_type=jax.ShapeDtypeStruct((NWORKERS, NBINS), jnp.int32),
        mesh=mesh,
        scratch_types=[
            pltpu.VMEM((NBINS,), jnp.int32),
            pltpu.VMEM((CHUNK,), jnp.int32),
            pltpu.VMEM((CHUNK,), jnp.int32),
            pltpu.SemaphoreType.DMA,
            pltpu.SemaphoreType.DMA,
        ],
        compiler_params=pltpu.CompilerParams(needs_layout_passes=False),
    )(_hist_body)
    return f(v32)


def _post_body(h_ref, out_ref):
    col = lax.broadcasted_iota(jnp.int32, (1, NBINS), 1)
    xcol = col.astype(jnp.float32)
    S = []
    for g in range(8):
        hc = jnp.sum(h_ref[g], axis=0, keepdims=True)
        mask = hc > 0
        mn = jnp.min(jnp.where(mask, col, NBINS - 1)).astype(jnp.float32)
        mx = jnp.max(jnp.where(mask, col, 0)).astype(jnp.float32)
        scale = jnp.float32(float(NBINS)) / (mx - mn)
        idx = jnp.floor((xcol - mn) * scale).astype(jnp.int32)
        idx = jnp.clip(idx, 0, NBINS - 1)
        ii = idx // LEVELS
        jj = idx - ii * LEVELS
        d = (ii - jj).astype(jnp.float32)
        S.append(jnp.sum(hc.astype(jnp.float32) * d * d))
    norm = jnp.float32(float(4 * PIX_PER_TENSOR))
    acc = jnp.float32(0.0)
    for k in range(4):
        acc = acc + jnp.abs(S[k] / norm - S[k + 4] / norm)
    out_ref[...] = jnp.broadcast_to(acc / 4.0, (1, 1))


def _post(hists):
    return pl.pallas_call(
        _post_body,
        out_shape=jax.ShapeDtypeStruct((1, 1), jnp.float32),
    )(hists)


def kernel(original, generated):
    v = _joint(original, generated)
    v32 = v.reshape(NWORKERS, PER_WORKER)
    hists = _hist(v32)
    loss = _post(hists.reshape(8, 4, NBINS))
    return loss[0, 0]


# speedup vs baseline: 199.1885x; 4.0966x over previous
import functools

import jax
import jax.numpy as jnp
from jax import lax
from jax.experimental import pallas as pl
from jax.experimental.pallas import tpu as pltpu
from jax.experimental.pallas import tpu_sc as plsc

LEVELS = 256
H = 512
W = 512
N_IMG = 8
NBINS = LEVELS * LEVELS
HR = 512
HC = 128
SHIFTS = ((1, 0), (1, 1), (0, 1), (-1, 1))

NWORKERS = 32
PIX_PER_TENSOR = N_IMG * H * W
WORDS_PER_WORKER = H * W
ROWS_PER_CHUNK = 32
NCHUNK = H // ROWS_PER_CHUNK
WORDS_PER_CHUNK = ROWS_PER_CHUNK * W


def _roll2(x, dx, dy):
    if dx == 1:
        x = jnp.concatenate([x[-1:, :], x[:-1, :]], axis=0)
    elif dx == -1:
        x = jnp.concatenate([x[1:, :], x[:1, :]], axis=0)
    if dy == 1:
        x = jnp.concatenate([x[:, -1:], x[:, :-1]], axis=1)
    return x


def _joint_body(o_ref, g_ref, v_ref):
    third = jnp.float32(1.0 / 3.0)
    for t, ref in enumerate((o_ref, g_ref)):
        grays = []
        for i in range(2):
            mean = (ref[i, 0] + ref[i, 1] + ref[i, 2]) * third
            grays.append(jnp.round(mean * float(LEVELS - 1)).astype(jnp.int32))
        for k, (dx, dy) in enumerate(SHIFTS):
            v0 = grays[0] * LEVELS + _roll2(grays[0], dx, dy)
            v1 = grays[1] * LEVELS + _roll2(grays[1], dx, dy)
            v_ref[t, k, 0] = v0 | (v1 << 16)


def _joint(original, generated):
    return pl.pallas_call(
        _joint_body,
        grid=(N_IMG // 2,),
        in_specs=[
            pl.BlockSpec((2, 3, H, W), lambda p: (p, 0, 0, 0)),
            pl.BlockSpec((2, 3, H, W), lambda p: (p, 0, 0, 0)),
        ],
        out_specs=pl.BlockSpec((2, 4, 1, H, W), lambda p: (0, 0, p, 0, 0)),
        out_shape=jax.ShapeDtypeStruct((2, 4, N_IMG // 2, H, W), jnp.int32),
    )(original, generated)


def _hist_body(v_hbm, out_hbm, hist, buf0, buf1, sem0, sem1):
    wid = lax.axis_index("s") * 2 + lax.axis_index("c")
    bufs = (buf0, buf1)
    sems = (sem0, sem1)

    zeros16 = jnp.zeros((16,), jnp.int32)
    ones16 = jnp.full((16,), 1, jnp.int32)
    mask16 = jnp.full((16,), 0xFFFF, jnp.int32)

    def zbody(i):
        hist[i >> 3, pl.ds((i & 7) * 16, 16)] = zeros16

    plsc.parallel_loop(0, (HR * HC) // 16, 1, unroll=8)(zbody)

    pltpu.async_copy(v_hbm.at[wid, pl.ds(0, ROWS_PER_CHUNK), :], bufs[0], sems[0])

    def make_scat(buf_s):
        def scat(j):
            word = buf_s[j >> 5, pl.ds((j & 31) * 16, 16)]
            lo = word & mask16
            hi = lax.shift_right_logical(word, 16)
            plsc.addupdate_scatter(
                hist, [lax.shift_right_logical(lo, 7), lo & (HC - 1)], ones16
            )
            plsc.addupdate_scatter(
                hist, [lax.shift_right_logical(hi, 7), hi & (HC - 1)], ones16
            )
        return scat

    def pair_body(p, carry):
        for s in range(2):
            c = p * 2 + s
            nxt = c + 1

            @pl.when(nxt < NCHUNK)
            def _():
                pltpu.async_copy(
                    v_hbm.at[wid, pl.ds(nxt * ROWS_PER_CHUNK, ROWS_PER_CHUNK), :],
                    bufs[1 - s],
                    sems[1 - s],
                )

            pltpu.make_async_copy(
                v_hbm.at[wid, pl.ds(c * ROWS_PER_CHUNK, ROWS_PER_CHUNK), :],
                bufs[s],
                sems[s],
            ).wait()

            plsc.parallel_loop(0, WORDS_PER_CHUNK // 16, 1, unroll=8)(
                make_scat(bufs[s])
            )
        return carry

    lax.fori_loop(0, NCHUNK // 2, pair_body, 0)

    pltpu.sync_copy(hist, out_hbm.at[wid])


def _hist(v32):
    mesh = plsc.VectorSubcoreMesh(core_axis_name="c", subcore_axis_name="s")
    f = functools.partial(
        pl.kernel,
        out_type=jax.ShapeDtypeStruct((NWORKERS, HR, HC), jnp.int32),
        mesh=mesh,
        scratch_types=[
            pltpu.VMEM((HR, HC), jnp.int32),
            pltpu.VMEM((ROWS_PER_CHUNK, W), jnp.int32),
            pltpu.VMEM((ROWS_PER_CHUNK, W), jnp.int32),
            pltpu.SemaphoreType.DMA,
            pltpu.SemaphoreType.DMA,
        ],
        compiler_params=pltpu.CompilerParams(needs_layout_passes=False),
    )(_hist_body)
    return f(v32)


def _post_body(h_ref, out_ref):
    rows = lax.broadcasted_iota(jnp.int32, (HR, HC), 0)
    lanes = lax.broadcasted_iota(jnp.int32, (HR, HC), 1)
    col = rows * HC + lanes
    xcol = col.astype(jnp.float32)
    S = []
    for g in range(8):
        hc = h_ref[4 * g] + h_ref[4 * g + 1] + h_ref[4 * g + 2] + h_ref[4 * g + 3]
        mask = hc > 0
        mn = jnp.min(jnp.where(mask, col, NBINS - 1)).astype(jnp.float32)
        mx = jnp.max(jnp.where(mask, col, 0)).astype(jnp.float32)
        scale = jnp.float32(float(NBINS)) / (mx - mn)
        idx = jnp.floor((xcol - mn) * scale).astype(jnp.int32)
        idx = jnp.clip(idx, 0, NBINS - 1)
        ii = idx // LEVELS
        jj = idx - ii * LEVELS
        d = (ii - jj).astype(jnp.float32)
        S.append(jnp.sum(hc.astype(jnp.float32) * d * d))
    norm = jnp.float32(float(4 * PIX_PER_TENSOR))
    acc = jnp.float32(0.0)
    for k in range(4):
        acc = acc + jnp.abs(S[k] / norm - S[k + 4] / norm)
    out_ref[...] = jnp.broadcast_to(acc / 4.0, (1, 1))


def _post(hists):
    return pl.pallas_call(
        _post_body,
        out_shape=jax.ShapeDtypeStruct((1, 1), jnp.float32),
    )(hists)


def kernel(original, generated):
    v = _joint(original, generated)
    v32 = v.reshape(NWORKERS, H, W)
    hists = _hist(v32)
    loss = _post(hists)
    return loss[0, 0]
